# baseline (speedup 1.0000x reference)
import functools

import jax
import jax.numpy as jnp
from jax import lax
from jax.experimental import pallas as pl
from jax.experimental.pallas import tpu as pltpu
from jax.experimental.pallas import tpu_sc as plsc

_NUM_CORES = 2
_NUM_SUBCORES = 16
_LANES = 16
_NW = _NUM_CORES * _NUM_SUBCORES

_B = 1024
_F = 128
_E = 256
_CHUNKS = _E // _LANES
_GROUPS = _F // _LANES
_ROWS_PER_W = _B // _NW
_ROW_WORDS = _F * _CHUNKS

_K = 1
_EMA_D = 0.95 ** (1.0 / 30000)


def _permute(x, idx):
    dn = lax.GatherDimensionNumbers(
        offset_dims=(), collapsed_slice_dims=(0,), start_index_map=(0,))
    return lax.gather(x, idx[:, None], dn, slice_sizes=(1,),
                      mode=lax.GatherScatterMode.PROMISE_IN_BOUNDS)


def _feature_sums(buf, g, perms, masks):
    vecs = []
    for f_in in range(_LANES):
        c0 = (g * _LANES + f_in) * _CHUNKS
        parts = [buf[c0 + e] for e in range(_CHUNKS)]
        while len(parts) > 1:
            parts = [parts[2 * j] + parts[2 * j + 1]
                     for j in range(len(parts) // 2)]
        vecs.append(parts[0])
    k = _LANES // 2
    s = 0
    while k >= 1:
        folded = [v + _permute(v, perms[s]) for v in vecs]
        vecs = [jnp.where(masks[s], folded[j + k], folded[j])
                for j in range(k)]
        k //= 2
        s += 1
    return vecs[0]


def _row_winner(buf, lane, perms, masks, zero_idx):
    bv = _feature_sums(buf, 0, perms, masks)
    bi = lane
    for g in range(1, _GROUPS):
        sums = _feature_sums(buf, g, perms, masks)
        better = sums > bv
        bv = jnp.where(better, sums, bv)
        bi = jnp.where(better, lane + g * _LANES, bi)
    kk, _ = plsc.sort_key_val(bv, bi, descending=True)
    mx = _permute(kk, zero_idx)
    cand = jnp.where(bv == mx, bi, jnp.full((_LANES,), _F, jnp.int32))
    cmin, _ = plsc.sort_key_val(cand, cand)
    return cmin


def _sc_body(r_hbm, out_hbm, buf0, buf1, hist, sem0, sem1):
    wid = lax.axis_index("c") * _NUM_SUBCORES + lax.axis_index("s")
    base = wid * _ROWS_PER_W
    last = base + _ROWS_PER_W - 1

    zero = jnp.zeros((_LANES,), jnp.float32)
    for g in range(_GROUPS):
        hist[pl.ds(g * _LANES, _LANES)] = zero

    lane = lax.iota(jnp.int32, _LANES)
    lane0 = lane == 0
    zero_idx = lane & 0
    ones_v = jnp.ones((_LANES,), jnp.float32)
    perms = []
    masks = []
    k = _LANES // 2
    while k >= 1:
        perms.append(lane ^ k)
        masks.append((lane & k) != 0)
        k //= 2

    pltpu.make_async_copy(r_hbm.at[base], buf0, sem0).start()
    pltpu.make_async_copy(r_hbm.at[base + 1], buf1, sem1).start()

    def pair_body(i, carry):
        r0 = base + 2 * i
        pltpu.make_async_copy(r_hbm.at[r0], buf0, sem0).wait()
        w0 = _row_winner(buf0, lane, perms, masks, zero_idx)
        plsc.addupdate_scatter(hist, [w0], ones_v, mask=lane0)
        pltpu.make_async_copy(
            r_hbm.at[jnp.minimum(r0 + 2, last)], buf0, sem0).start()
        pltpu.make_async_copy(r_hbm.at[r0 + 1], buf1, sem1).wait()
        w1 = _row_winner(buf1, lane, perms, masks, zero_idx)
        plsc.addupdate_scatter(hist, [w1], ones_v, mask=lane0)
        pltpu.make_async_copy(
            r_hbm.at[jnp.minimum(r0 + 3, last)], buf1, sem1).start()
        return carry

    lax.fori_loop(0, _ROWS_PER_W // 2, pair_body, 0)

    pltpu.make_async_copy(r_hbm.at[last], buf0, sem0).wait()
    pltpu.make_async_copy(r_hbm.at[last], buf1, sem1).wait()

    pltpu.sync_copy(hist, out_hbm.at[wid])


_sc_win_hist = functools.partial(
    pl.kernel,
    out_type=jax.ShapeDtypeStruct((_NW, _F), jnp.float32),
    mesh=plsc.VectorSubcoreMesh(
        core_axis_name="c", subcore_axis_name="s",
        num_cores=_NUM_CORES, num_subcores=_NUM_SUBCORES),
    scratch_types=[
        pltpu.VMEM((_ROW_WORDS, _LANES), jnp.float32),
        pltpu.VMEM((_ROW_WORDS, _LANES), jnp.float32),
        pltpu.VMEM((_F,), jnp.float32),
        pltpu.SemaphoreType.DMA,
        pltpu.SemaphoreType.DMA,
    ],
    compiler_params=pltpu.CompilerParams(
        needs_layout_passes=False, use_tc_tiling_on_sc=False),
)(_sc_body)


def _finish_body(p_ref, freq_ref, boost_ref):
    counts = jnp.sum(p_ref[...], axis=0)
    freq = counts * (1.0 / float(_K * _B))
    freq_ref[...] = _EMA_D * freq + (1.0 - _EMA_D) * freq
    boost_ref[...] = jnp.ones((_F,), jnp.float32)


_finish = pl.pallas_call(
    _finish_body,
    out_shape=(
        jax.ShapeDtypeStruct((_F,), jnp.float32),
        jax.ShapeDtypeStruct((_F,), jnp.float32),
    ),
)


def kernel(R):
    r = R.reshape(_B, _ROW_WORDS, _LANES)
    partials = _sc_win_hist(r)
    freq_ema, boosting_weights = _finish(partials)
    return freq_ema, boosting_weights


# speedup vs baseline: 1.1555x; 1.1555x over previous
import functools

import jax
import jax.numpy as jnp
from jax import lax
from jax.experimental import pallas as pl
from jax.experimental.pallas import tpu as pltpu
from jax.experimental.pallas import tpu_sc as plsc

_NUM_CORES = 2
_NUM_SUBCORES = 16
_LANES = 16
_NW = _NUM_CORES * _NUM_SUBCORES

_B = 1024
_F = 128
_E = 256
_CHUNKS = _E // _LANES
_GROUPS = _F // _LANES
_ROWS_PER_W = _B // _NW

_K = 1
_EMA_D = 0.95 ** (1.0 / 30000)


def _permute(x, idx):
    dn = lax.GatherDimensionNumbers(
        offset_dims=(), collapsed_slice_dims=(0,), start_index_map=(0,))
    return lax.gather(x, idx[:, None], dn, slice_sizes=(1,),
                      mode=lax.GatherScatterMode.PROMISE_IN_BOUNDS)


def _row_winner(buf, stage, lane, perms, masks, zero_idx):
    def group_body(g, carry):
        bv, bi = carry

        def feat_body(f_in, c):
            f = g * _LANES + f_in
            parts = [buf[f, e] for e in range(_CHUNKS)]
            while len(parts) > 1:
                parts = [parts[2 * j] + parts[2 * j + 1]
                         for j in range(len(parts) // 2)]
            stage[f_in] = parts[0]
            return c

        lax.fori_loop(0, _LANES, feat_body, 0)
        vecs = [stage[j] for j in range(_LANES)]
        k = _LANES // 2
        s = 0
        while k >= 1:
            folded = [v + _permute(v, perms[s]) for v in vecs]
            vecs = [jnp.where(masks[s], folded[j + k], folded[j])
                    for j in range(k)]
            k //= 2
            s += 1
        sums = vecs[0]
        fidx = lane + g * _LANES
        better = sums > bv
        return jnp.where(better, sums, bv), jnp.where(better, fidx, bi)

    neg = jnp.full((_LANES,), -jnp.inf, jnp.float32)
    bv, bi = lax.fori_loop(0, _GROUPS, group_body, (neg, lane))
    kk, _ = plsc.sort_key_val(bv, bi, descending=True)
    mx = _permute(kk, zero_idx)
    cand = jnp.where(bv == mx, bi, jnp.full((_LANES,), _F, jnp.int32))
    cmin, _ = plsc.sort_key_val(cand, cand)
    return cmin


def _sc_body(r_hbm, out_hbm, buf0, buf1, stage, hist, sem0, sem1):
    wid = lax.axis_index("c") * _NUM_SUBCORES + lax.axis_index("s")
    base = wid * _ROWS_PER_W
    last = base + _ROWS_PER_W - 1

    zero = jnp.zeros((_LANES,), jnp.float32)
    for g in range(_GROUPS):
        hist[pl.ds(g * _LANES, _LANES)] = zero

    lane = lax.iota(jnp.int32, _LANES)
    lane0 = lane == 0
    zero_idx = lane & 0
    ones_v = jnp.ones((_LANES,), jnp.float32)
    perms = []
    masks = []
    k = _LANES // 2
    while k >= 1:
        perms.append(lane ^ k)
        masks.append((lane & k) != 0)
        k //= 2

    pltpu.make_async_copy(r_hbm.at[base], buf0, sem0).start()
    pltpu.make_async_copy(r_hbm.at[base + 1], buf1, sem1).start()

    def pair_body(i, carry):
        r0 = base + 2 * i
        pltpu.make_async_copy(r_hbm.at[r0], buf0, sem0).wait()
        w0 = _row_winner(buf0, stage, lane, perms, masks, zero_idx)
        plsc.addupdate_scatter(hist, [w0], ones_v, mask=lane0)
        pltpu.make_async_copy(
            r_hbm.at[jnp.minimum(r0 + 2, last)], buf0, sem0).start()
        pltpu.make_async_copy(r_hbm.at[r0 + 1], buf1, sem1).wait()
        w1 = _row_winner(buf1, stage, lane, perms, masks, zero_idx)
        plsc.addupdate_scatter(hist, [w1], ones_v, mask=lane0)
        pltpu.make_async_copy(
            r_hbm.at[jnp.minimum(r0 + 3, last)], buf1, sem1).start()
        return carry

    lax.fori_loop(0, _ROWS_PER_W // 2, pair_body, 0)

    pltpu.make_async_copy(r_hbm.at[last], buf0, sem0).wait()
    pltpu.make_async_copy(r_hbm.at[last], buf1, sem1).wait()

    pltpu.sync_copy(hist, out_hbm.at[wid])


_sc_win_hist = functools.partial(
    pl.kernel,
    out_type=jax.ShapeDtypeStruct((_NW, _F), jnp.float32),
    mesh=plsc.VectorSubcoreMesh(
        core_axis_name="c", subcore_axis_name="s",
        num_cores=_NUM_CORES, num_subcores=_NUM_SUBCORES),
    scratch_types=[
        pltpu.VMEM((_F, _CHUNKS, _LANES), jnp.float32),
        pltpu.VMEM((_F, _CHUNKS, _LANES), jnp.float32),
        pltpu.VMEM((_LANES, _LANES), jnp.float32),
        pltpu.VMEM((_F,), jnp.float32),
        pltpu.SemaphoreType.DMA,
        pltpu.SemaphoreType.DMA,
    ],
    compiler_params=pltpu.CompilerParams(
        needs_layout_passes=False, use_tc_tiling_on_sc=False),
)(_sc_body)


def _finish_body(p_ref, freq_ref, boost_ref):
    counts = jnp.sum(p_ref[...], axis=0)
    freq = counts * (1.0 / float(_K * _B))
    freq_ref[...] = _EMA_D * freq + (1.0 - _EMA_D) * freq
    boost_ref[...] = jnp.ones((_F,), jnp.float32)


_finish = pl.pallas_call(
    _finish_body,
    out_shape=(
        jax.ShapeDtypeStruct((_F,), jnp.float32),
        jax.ShapeDtypeStruct((_F,), jnp.float32),
    ),
)


def kernel(R):
    partials = _sc_win_hist(R)
    freq_ema, boosting_weights = _finish(partials)
    return freq_ema, boosting_weights


# speedup vs baseline: 3.1850x; 2.7563x over previous
import functools

import jax
import jax.numpy as jnp
from jax import lax
from jax.experimental import pallas as pl
from jax.experimental.pallas import tpu as pltpu
from jax.experimental.pallas import tpu_sc as plsc

_NUM_CORES = 2
_NUM_SUBCORES = 16
_LANES = 16
_NW = _NUM_CORES * _NUM_SUBCORES

_B = 1024
_F = 128
_E = 256
_CHUNKS = _E // _LANES
_GROUPS = _F // _LANES
_ROWS_PER_W = _B // _NW

_K = 1
_EMA_D = 0.95 ** (1.0 / 30000)


def _permute(x, idx):
    dn = lax.GatherDimensionNumbers(
        offset_dims=(), collapsed_slice_dims=(0,), start_index_map=(0,))
    return lax.gather(x, idx[:, None], dn, slice_sizes=(1,),
                      mode=lax.GatherScatterMode.PROMISE_IN_BOUNDS)


def _row_winner(buf, stage, lane, perms, masks, zero_idx):
    def group_body(g, carry):
        bv, bi = carry

        def feat_body(f_in, c):
            f = g * _LANES + f_in
            parts = [buf[2 * f + (e // 8), pl.ds((e % 8) * _LANES, _LANES)]
                     for e in range(_CHUNKS)]
            while len(parts) > 1:
                parts = [parts[2 * j] + parts[2 * j + 1]
                         for j in range(len(parts) // 2)]
            stage[f_in] = parts[0]
            return c

        lax.fori_loop(0, _LANES, feat_body, 0)
        vecs = [stage[j] for j in range(_LANES)]
        k = _LANES // 2
        s = 0
        while k >= 1:
            folded = [v + _permute(v, perms[s]) for v in vecs]
            vecs = [jnp.where(masks[s], folded[j + k], folded[j])
                    for j in range(k)]
            k //= 2
            s += 1
        sums = vecs[0]
        fidx = lane + g * _LANES
        better = sums > bv
        return jnp.where(better, sums, bv), jnp.where(better, fidx, bi)

    neg = jnp.full((_LANES,), -jnp.inf, jnp.float32)
    bv, bi = lax.fori_loop(0, _GROUPS, group_body, (neg, lane))
    kk, _ = plsc.sort_key_val(bv, bi, descending=True)
    mx = _permute(kk, zero_idx)
    cand = jnp.where(bv == mx, bi, jnp.full((_LANES,), _F, jnp.int32))
    cmin, _ = plsc.sort_key_val(cand, cand)
    return cmin


def _sc_body(r_hbm, out_hbm, buf0, buf1, stage, hist, sem0, sem1):
    wid = lax.axis_index("c") * _NUM_SUBCORES + lax.axis_index("s")
    base = wid * _ROWS_PER_W
    last = base + _ROWS_PER_W - 1

    zero = jnp.zeros((_LANES,), jnp.float32)
    for g in range(_GROUPS):
        hist[pl.ds(g * _LANES, _LANES)] = zero

    lane = lax.iota(jnp.int32, _LANES)
    lane0 = lane == 0
    zero_idx = lane & 0
    ones_v = jnp.ones((_LANES,), jnp.float32)
    perms = []
    masks = []
    k = _LANES // 2
    while k >= 1:
        perms.append(lane ^ k)
        masks.append((lane & k) != 0)
        k //= 2

    pltpu.make_async_copy(r_hbm.at[base], buf0, sem0).start()
    pltpu.make_async_copy(r_hbm.at[base + 1], buf1, sem1).start()

    def pair_body(i, carry):
        r0 = base + 2 * i
        pltpu.make_async_copy(r_hbm.at[r0], buf0, sem0).wait()
        w0 = _row_winner(buf0, stage, lane, perms, masks, zero_idx)
        plsc.addupdate_scatter(hist, [w0], ones_v, mask=lane0)
        pltpu.make_async_copy(
            r_hbm.at[jnp.minimum(r0 + 2, last)], buf0, sem0).start()
        pltpu.make_async_copy(r_hbm.at[r0 + 1], buf1, sem1).wait()
        w1 = _row_winner(buf1, stage, lane, perms, masks, zero_idx)
        plsc.addupdate_scatter(hist, [w1], ones_v, mask=lane0)
        pltpu.make_async_copy(
            r_hbm.at[jnp.minimum(r0 + 3, last)], buf1, sem1).start()
        return carry

    lax.fori_loop(0, _ROWS_PER_W // 2, pair_body, 0)

    pltpu.make_async_copy(r_hbm.at[last], buf0, sem0).wait()
    pltpu.make_async_copy(r_hbm.at[last], buf1, sem1).wait()

    pltpu.sync_copy(hist, out_hbm.at[wid])


_sc_win_hist = functools.partial(
    pl.kernel,
    out_type=jax.ShapeDtypeStruct((_NW, _F), jnp.float32),
    mesh=plsc.VectorSubcoreMesh(
        core_axis_name="c", subcore_axis_name="s",
        num_cores=_NUM_CORES, num_subcores=_NUM_SUBCORES),
    scratch_types=[
        pltpu.VMEM((2 * _F, 128), jnp.float32),
        pltpu.VMEM((2 * _F, 128), jnp.float32),
        pltpu.VMEM((_LANES, _LANES), jnp.float32),
        pltpu.VMEM((_F,), jnp.float32),
        pltpu.SemaphoreType.DMA,
        pltpu.SemaphoreType.DMA,
    ],
    compiler_params=pltpu.CompilerParams(
        needs_layout_passes=False, use_tc_tiling_on_sc=True),
)(_sc_body)


def _finish_body(p_ref, freq_ref, boost_ref):
    counts = jnp.sum(p_ref[...], axis=0)
    freq = counts * (1.0 / float(_K * _B))
    freq_ref[...] = _EMA_D * freq + (1.0 - _EMA_D) * freq
    boost_ref[...] = jnp.ones((_F,), jnp.float32)


_finish = pl.pallas_call(
    _finish_body,
    out_shape=(
        jax.ShapeDtypeStruct((_F,), jnp.float32),
        jax.ShapeDtypeStruct((_F,), jnp.float32),
    ),
)


def kernel(R):
    r = R.reshape(_B, 2 * _F, 128)
    partials = _sc_win_hist(r)
    freq_ema, boosting_weights = _finish(partials)
    return freq_ema, boosting_weights
